# Initial kernel scaffold; baseline (speedup 1.0000x reference)
import jax
import jax.numpy as jnp
from jax.experimental import pallas as pl


def kernel(x0, x1, x2, x3, x4, params):
    raise NotImplementedError("write your pallas kernel here")


# speedup vs baseline: 24.3513x; 24.3513x over previous
import functools

import jax
import jax.numpy as jnp
import numpy as np
from jax.experimental import pallas as pl
from jax.experimental.pallas import tpu as pltpu

_NUM_ANCHORS = 3
_RATIOS = np.array([0.5, 1.0, 2.0])
_BASE_SIZE = 8.0
_STD_AB = 0.5


def _conv_decode_kernel(xp_ref, w_ref, b_ref, rw_ref, rb_ref, ow_ref, ob_ref,
                        out_ref, t_scr, *, W, TH, WP):
    i = pl.program_id(0)
    n = TH * W

    def conv_row(r):
        rows = []
        for kh in range(3):
            xrow = xp_ref[:, pl.ds(r + kh, 1), :].reshape(256, WP)
            for kw in range(3):
                rows.append(xrow[:, kw:kw + W])
        a = jnp.concatenate(rows, axis=0)
        acc = jax.lax.dot_general(
            w_ref[...], a, (((1,), (0,)), ((), ())),
            preferred_element_type=jnp.float32)
        return acc + b_ref[:, :]

    if W >= 128:
        def row_body(r, carry):
            t_scr[:, pl.ds(r * W, W)] = conv_row(r)
            return carry

        jax.lax.fori_loop(0, TH, row_body, 0, unroll=False)
        t = t_scr[...]
    else:
        t = jnp.concatenate([conv_row(r) for r in range(TH)], axis=1)

    off = jax.lax.dot_general(rw_ref[...], t, (((1,), (0,)), ((), ())),
                              preferred_element_type=jnp.float32) + rb_ref[:, :]
    obj = jax.lax.dot_general(ow_ref[...], t, (((1,), (0,)), ((), ())),
                              preferred_element_type=jnp.float32) + ob_ref[:, :]

    ii = jax.lax.broadcasted_iota(jnp.int32, (1, n), 1)
    cx = (ii % W).astype(jnp.float32) + 0.5
    cy = (ii // W).astype(jnp.float32) + (i * TH).astype(jnp.float32) + 0.5

    ws_all = _BASE_SIZE * np.sqrt(_RATIOS)
    hs_all = _BASE_SIZE / np.sqrt(_RATIOS)

    for a_i in range(_NUM_ANCHORS):
        dx = off[6 * a_i + 0:6 * a_i + 1, :]
        dy = off[6 * a_i + 1:6 * a_i + 2, :]
        dw = off[6 * a_i + 2:6 * a_i + 3, :]
        dh = off[6 * a_i + 3:6 * a_i + 4, :]
        da = off[6 * a_i + 4:6 * a_i + 5, :] * _STD_AB
        db = off[6 * a_i + 5:6 * a_i + 6, :] * _STD_AB
        aw = float(ws_all[a_i])
        ah = float(hs_all[a_i])
        w_ = aw * jnp.exp(jnp.clip(dw, -8.0, 8.0))
        h_ = ah * jnp.exp(jnp.clip(dh, -8.0, 8.0))
        px = cx + dx * aw
        py = cy + dy * ah
        dal = da * w_
        dbe = db * h_
        v1x = px + dal
        v1y = py - h_ / 2.0
        v2x = px + w_ / 2.0
        v2y = py + dbe
        v3x = px - dal
        v3y = py + h_ / 2.0
        v4x = px - w_ / 2.0
        v4y = py - dbe
        x1 = jnp.minimum(jnp.minimum(v1x, v2x), jnp.minimum(v3x, v4x))
        y1 = jnp.minimum(jnp.minimum(v1y, v2y), jnp.minimum(v3y, v4y))
        x2 = jnp.maximum(jnp.maximum(v1x, v2x), jnp.maximum(v3x, v4x))
        y2 = jnp.maximum(jnp.maximum(v1y, v2y), jnp.maximum(v3y, v4y))
        out_ref[a_i, :, :] = jnp.concatenate(
            [v1x, v1y, v2x, v2y, v3x, v3y, v4x, v4y,
             x1, y1, x2, y2, obj[a_i:a_i + 1, :]], axis=0)


def _run_level(x, p, H, W):
    if H * W <= 4096:
        TH = H
    else:
        TH = max(8, 4096 // W)
        while H % TH:
            TH -= 1
    n_tiles = H // TH
    WP = max(128, ((W + 2 + 127) // 128) * 128)

    BH = H + 2 if n_tiles == 1 else ((TH + 2 + 7) // 8) * 8
    HP = H + 2 if n_tiles == 1 else (n_tiles - 1) * TH + BH
    xp = jnp.zeros((256, HP, WP), jnp.float32)
    xp = jax.lax.dynamic_update_slice(xp, x[0], (0, 1, 1))
    ow_pad = jnp.zeros((8, 256), jnp.float32).at[:3].set(p['obj_w'][:, :, 0, 0])
    ob_pad = jnp.zeros((8, 1), jnp.float32).at[:3, 0].set(p['obj_b'])

    if n_tiles == 1:
        x_spec = pl.BlockSpec((256, BH, WP), lambda i: (0, 0, 0))
    else:
        x_spec = pl.BlockSpec((pl.Element(256), pl.Element(BH), pl.Element(WP)),
                              lambda i: (0, i * TH, 0))

    out = pl.pallas_call(
        functools.partial(_conv_decode_kernel, W=W, TH=TH, WP=WP),
        grid=(n_tiles,),
        in_specs=[
            x_spec,
            pl.BlockSpec((256, 2304), lambda i: (0, 0)),
            pl.BlockSpec((256, 1), lambda i: (0, 0)),
            pl.BlockSpec((18, 256), lambda i: (0, 0)),
            pl.BlockSpec((18, 1), lambda i: (0, 0)),
            pl.BlockSpec((8, 256), lambda i: (0, 0)),
            pl.BlockSpec((8, 1), lambda i: (0, 0)),
        ],
        out_specs=pl.BlockSpec((3, 13, TH * W), lambda i: (0, 0, i)),
        out_shape=jax.ShapeDtypeStruct((3, 13, H * W), jnp.float32),
        scratch_shapes=[pltpu.VMEM((256, TH * W), jnp.float32)],
    )(xp, p['conv_w'].transpose(0, 2, 3, 1).reshape(256, 2304),
      p['conv_b'][:, None], p['reg_w'][:, :, 0, 0],
      p['reg_b'][:, None], ow_pad, ob_pad)

    obj = out[:, 12, :].reshape(-1)
    preds = jnp.moveaxis(out[:, :8, :], 1, 2).reshape(-1, 4, 2)
    hbb = jnp.moveaxis(out[:, 8:12, :], 1, 2).reshape(-1, 4)
    return obj, preds, hbb


_NMS_PAD = 2048


def _nms_kernel(k_ref, bs_ref, boxes_ref, keep_ref):
    NPAD = _NMS_PAD
    x1 = boxes_ref[0:1, :]
    y1 = boxes_ref[1:2, :]
    x2 = boxes_ref[2:3, :]
    y2 = boxes_ref[3:4, :]
    areas = jnp.maximum(x2 - x1, 0.0) * jnp.maximum(y2 - y1, 0.0)
    idx = jax.lax.broadcasted_iota(jnp.int32, (1, NPAD), 1).astype(jnp.float32)

    def body(i, keep):
        fi = i.astype(jnp.float32)
        ki = jnp.sum(keep * (idx == fi).astype(jnp.float32))
        bx1 = bs_ref[0, i]
        by1 = bs_ref[1, i]
        bx2 = bs_ref[2, i]
        by2 = bs_ref[3, i]
        ar_i = jnp.maximum(bx2 - bx1, 0.0) * jnp.maximum(by2 - by1, 0.0)
        xx1 = jnp.maximum(x1, bx1)
        yy1 = jnp.maximum(y1, by1)
        xx2 = jnp.minimum(x2, bx2)
        yy2 = jnp.minimum(y2, by2)
        inter = jnp.maximum(xx2 - xx1, 0.0) * jnp.maximum(yy2 - yy1, 0.0)
        iou = inter / (areas + ar_i - inter + 1e-9)
        sup = ((iou > 0.5) & (idx > fi)).astype(jnp.float32)
        return keep * (1.0 - ki * sup)

    keep_ref[...] = jax.lax.fori_loop(0, k_ref[0], body,
                                      jnp.ones((1, NPAD), jnp.float32))


def _nms(boxes, k):
    bpad = jnp.zeros((_NMS_PAD, 4), jnp.float32).at[:k].set(boxes).T
    keep = pl.pallas_call(
        _nms_kernel,
        in_specs=[
            pl.BlockSpec(memory_space=pltpu.SMEM),
            pl.BlockSpec(memory_space=pltpu.SMEM),
            pl.BlockSpec((4, _NMS_PAD), lambda: (0, 0)),
        ],
        out_shape=jax.ShapeDtypeStruct((1, _NMS_PAD), jnp.float32),
    )(jnp.array([k], jnp.int32), bpad, bpad)
    return keep[0, :k] > 0.5


def kernel(x0, x1, x2, x3, x4, params):
    feats = [x0, x1, x2, x3, x4]
    merged_props = []
    merged_scores = []
    for lvl in range(5):
        x = feats[lvl]
        p = params[str(lvl)]
        _, _, H, W = x.shape
        obj, preds, hbb = _run_level(x, p, H, W)
        k = min(2000, obj.shape[0])
        scores, ti = jax.lax.top_k(obj, k)
        keep = _nms(hbb[ti], k)
        s = jnp.where(keep, scores, -1e9)
        merged_props.append(preds[ti][None])
        merged_scores.append(s[None])
    props = jnp.concatenate(merged_props, axis=1)
    scores = jnp.concatenate(merged_scores, axis=1)
    kk = min(1000, scores.shape[1])
    _, ti = jax.lax.top_k(scores[0], kk)
    return props[:, ti]


# speedup vs baseline: 29.2275x; 1.2002x over previous
import functools

import jax
import jax.numpy as jnp
import numpy as np
from jax.experimental import pallas as pl
from jax.experimental.pallas import tpu as pltpu

_NUM_ANCHORS = 3
_RATIOS = np.array([0.5, 1.0, 2.0])
_BASE_SIZE = 8.0
_STD_AB = 0.5


def _conv_decode_kernel(xp_ref, w_ref, b_ref, rw_ref, rb_ref, ow_ref, ob_ref,
                        out_ref, t_scr, *, W, TH, WP):
    i = pl.program_id(0)
    n = TH * W

    def conv_rows(r, RP):
        xr = [xp_ref[:, pl.ds(r + j, 1), :].reshape(256, WP)
              for j in range(RP + 2)]
        ks = []
        for kh in range(3):
            for kw in range(3):
                if RP == 1:
                    ks.append(xr[kh][:, kw:kw + W])
                else:
                    ks.append(jnp.concatenate(
                        [xr[j + kh][:, kw:kw + W] for j in range(RP)], axis=1))
        a = jnp.concatenate(ks, axis=0)
        acc = jax.lax.dot_general(
            w_ref[...], a, (((1,), (0,)), ((), ())),
            preferred_element_type=jnp.float32)
        return acc + b_ref[:, :]

    if W >= 128:
        RP = 4
        def row_body(it, carry):
            r = it * RP
            t_scr[:, pl.ds(r * W, RP * W)] = conv_rows(r, RP)
            return carry

        jax.lax.fori_loop(0, TH // RP, row_body, 0, unroll=False)
        t = t_scr[...]
    else:
        RP = min(TH, max(1, 512 // W))
        t = jnp.concatenate(
            [conv_rows(r * RP, RP) for r in range(TH // RP)], axis=1)

    off = jax.lax.dot_general(rw_ref[...], t, (((1,), (0,)), ((), ())),
                              preferred_element_type=jnp.float32) + rb_ref[:, :]
    obj = jax.lax.dot_general(ow_ref[...], t, (((1,), (0,)), ((), ())),
                              preferred_element_type=jnp.float32) + ob_ref[:, :]

    ii = jax.lax.broadcasted_iota(jnp.int32, (1, n), 1)
    cx = (ii % W).astype(jnp.float32) + 0.5
    cy = (ii // W).astype(jnp.float32) + (i * TH).astype(jnp.float32) + 0.5

    ws_all = _BASE_SIZE * np.sqrt(_RATIOS)
    hs_all = _BASE_SIZE / np.sqrt(_RATIOS)

    for a_i in range(_NUM_ANCHORS):
        dx = off[6 * a_i + 0:6 * a_i + 1, :]
        dy = off[6 * a_i + 1:6 * a_i + 2, :]
        dw = off[6 * a_i + 2:6 * a_i + 3, :]
        dh = off[6 * a_i + 3:6 * a_i + 4, :]
        da = off[6 * a_i + 4:6 * a_i + 5, :] * _STD_AB
        db = off[6 * a_i + 5:6 * a_i + 6, :] * _STD_AB
        aw = float(ws_all[a_i])
        ah = float(hs_all[a_i])
        w_ = aw * jnp.exp(jnp.clip(dw, -8.0, 8.0))
        h_ = ah * jnp.exp(jnp.clip(dh, -8.0, 8.0))
        px = cx + dx * aw
        py = cy + dy * ah
        dal = da * w_
        dbe = db * h_
        v1x = px + dal
        v1y = py - h_ / 2.0
        v2x = px + w_ / 2.0
        v2y = py + dbe
        v3x = px - dal
        v3y = py + h_ / 2.0
        v4x = px - w_ / 2.0
        v4y = py - dbe
        x1 = jnp.minimum(jnp.minimum(v1x, v2x), jnp.minimum(v3x, v4x))
        y1 = jnp.minimum(jnp.minimum(v1y, v2y), jnp.minimum(v3y, v4y))
        x2 = jnp.maximum(jnp.maximum(v1x, v2x), jnp.maximum(v3x, v4x))
        y2 = jnp.maximum(jnp.maximum(v1y, v2y), jnp.maximum(v3y, v4y))
        out_ref[a_i, :, :] = jnp.concatenate(
            [v1x, v1y, v2x, v2y, v3x, v3y, v4x, v4y,
             x1, y1, x2, y2, obj[a_i:a_i + 1, :]], axis=0)


def _run_level(x, p, H, W):
    if H * W <= 4096:
        TH = H
    else:
        TH = max(8, 4096 // W)
        while H % TH:
            TH -= 1
    n_tiles = H // TH
    WP = max(128, ((W + 2 + 127) // 128) * 128)

    BH = H + 2 if n_tiles == 1 else ((TH + 2 + 7) // 8) * 8
    HP = H + 2 if n_tiles == 1 else (n_tiles - 1) * TH + BH
    xp = jnp.zeros((256, HP, WP), jnp.float32)
    xp = jax.lax.dynamic_update_slice(xp, x[0], (0, 1, 1))
    ow_pad = jnp.zeros((8, 256), jnp.float32).at[:3].set(p['obj_w'][:, :, 0, 0])
    ob_pad = jnp.zeros((8, 1), jnp.float32).at[:3, 0].set(p['obj_b'])

    if n_tiles == 1:
        x_spec = pl.BlockSpec((256, BH, WP), lambda i: (0, 0, 0))
    else:
        x_spec = pl.BlockSpec((pl.Element(256), pl.Element(BH), pl.Element(WP)),
                              lambda i: (0, i * TH, 0))

    out = pl.pallas_call(
        functools.partial(_conv_decode_kernel, W=W, TH=TH, WP=WP),
        grid=(n_tiles,),
        in_specs=[
            x_spec,
            pl.BlockSpec((256, 2304), lambda i: (0, 0)),
            pl.BlockSpec((256, 1), lambda i: (0, 0)),
            pl.BlockSpec((18, 256), lambda i: (0, 0)),
            pl.BlockSpec((18, 1), lambda i: (0, 0)),
            pl.BlockSpec((8, 256), lambda i: (0, 0)),
            pl.BlockSpec((8, 1), lambda i: (0, 0)),
        ],
        out_specs=pl.BlockSpec((3, 13, TH * W), lambda i: (0, 0, i)),
        out_shape=jax.ShapeDtypeStruct((3, 13, H * W), jnp.float32),
        scratch_shapes=[pltpu.VMEM((256, TH * W), jnp.float32)],
    )(xp, p['conv_w'].transpose(0, 2, 3, 1).reshape(256, 2304),
      p['conv_b'][:, None], p['reg_w'][:, :, 0, 0],
      p['reg_b'][:, None], ow_pad, ob_pad)

    obj = out[:, 12, :].reshape(-1)
    preds = jnp.moveaxis(out[:, :8, :], 1, 2).reshape(-1, 4, 2)
    hbb = jnp.moveaxis(out[:, 8:12, :], 1, 2).reshape(-1, 4)
    return obj, preds, hbb


_NMS_PAD = 2048


_NMS_R = 16
_NMS_C = 128


def _nms_kernel(k_ref, bs_ref, boxes_ref, keep_ref):
    R, C = _NMS_R, _NMS_C
    x1 = boxes_ref[0, :, :]
    y1 = boxes_ref[1, :, :]
    x2 = boxes_ref[2, :, :]
    y2 = boxes_ref[3, :, :]
    areas = jnp.maximum(x2 - x1, 0.0) * jnp.maximum(y2 - y1, 0.0)
    idx = (jax.lax.broadcasted_iota(jnp.int32, (R, C), 0) * C
           + jax.lax.broadcasted_iota(jnp.int32, (R, C), 1)).astype(jnp.float32)

    def body(i, keep):
        fi = i.astype(jnp.float32)
        ki = jnp.sum(keep * (idx == fi).astype(jnp.float32))
        bx1 = bs_ref[0, i]
        by1 = bs_ref[1, i]
        bx2 = bs_ref[2, i]
        by2 = bs_ref[3, i]
        ar_i = jnp.maximum(bx2 - bx1, 0.0) * jnp.maximum(by2 - by1, 0.0)
        xx1 = jnp.maximum(x1, bx1)
        yy1 = jnp.maximum(y1, by1)
        xx2 = jnp.minimum(x2, bx2)
        yy2 = jnp.minimum(y2, by2)
        inter = jnp.maximum(xx2 - xx1, 0.0) * jnp.maximum(yy2 - yy1, 0.0)
        iou = inter / (areas + ar_i - inter + 1e-9)
        sup = ((iou > 0.5) & (idx > fi)).astype(jnp.float32)
        return keep * (1.0 - ki * sup)

    keep_ref[...] = jax.lax.fori_loop(0, k_ref[0], body,
                                      jnp.ones((R, C), jnp.float32))


def _nms(boxes, k):
    NPAD = _NMS_R * _NMS_C
    bpad = jnp.zeros((NPAD, 4), jnp.float32).at[:k].set(boxes).T
    keep = pl.pallas_call(
        _nms_kernel,
        in_specs=[
            pl.BlockSpec(memory_space=pltpu.SMEM),
            pl.BlockSpec(memory_space=pltpu.SMEM),
            pl.BlockSpec((4, _NMS_R, _NMS_C), lambda: (0, 0, 0)),
        ],
        out_shape=jax.ShapeDtypeStruct((_NMS_R, _NMS_C), jnp.float32),
    )(jnp.array([k], jnp.int32), bpad, bpad.reshape(4, _NMS_R, _NMS_C))
    return keep.reshape(-1)[:k] > 0.5


def kernel(x0, x1, x2, x3, x4, params):
    feats = [x0, x1, x2, x3, x4]
    merged_props = []
    merged_scores = []
    for lvl in range(5):
        x = feats[lvl]
        p = params[str(lvl)]
        _, _, H, W = x.shape
        obj, preds, hbb = _run_level(x, p, H, W)
        k = min(2000, obj.shape[0])
        scores, ti = jax.lax.top_k(obj, k)
        keep = _nms(hbb[ti], k)
        s = jnp.where(keep, scores, -1e9)
        merged_props.append(preds[ti][None])
        merged_scores.append(s[None])
    props = jnp.concatenate(merged_props, axis=1)
    scores = jnp.concatenate(merged_scores, axis=1)
    kk = min(1000, scores.shape[1])
    _, ti = jax.lax.top_k(scores[0], kk)
    return props[:, ti]


# speedup vs baseline: 30.4907x; 1.0432x over previous
import functools

import jax
import jax.numpy as jnp
import numpy as np
from jax.experimental import pallas as pl
from jax.experimental.pallas import tpu as pltpu

_NUM_ANCHORS = 3
_RATIOS = np.array([0.5, 1.0, 2.0])
_BASE_SIZE = 8.0
_STD_AB = 0.5


def _conv_decode_kernel(xp_ref, w_ref, b_ref, rw_ref, rb_ref, ow_ref, ob_ref,
                        out_ref, t_scr, *, W, TH, WP):
    i = pl.program_id(0)
    n = TH * W

    def conv_rows(r, RP):
        xr = [xp_ref[:, pl.ds(r + j, 1), :].reshape(256, WP)
              for j in range(RP + 2)]
        ks = []
        for kh in range(3):
            for kw in range(3):
                if RP == 1:
                    ks.append(xr[kh][:, kw:kw + W])
                else:
                    ks.append(jnp.concatenate(
                        [xr[j + kh][:, kw:kw + W] for j in range(RP)], axis=1))
        a = jnp.concatenate(ks, axis=0)
        acc = jax.lax.dot_general(
            w_ref[...], a, (((1,), (0,)), ((), ())),
            preferred_element_type=jnp.float32)
        return acc + b_ref[:, :]

    if W >= 128:
        RP = 4
        def row_body(it, carry):
            r = it * RP
            t_scr[:, pl.ds(r * W, RP * W)] = conv_rows(r, RP)
            return carry

        jax.lax.fori_loop(0, TH // RP, row_body, 0, unroll=False)
        t = t_scr[...]
    else:
        RP = min(TH, max(1, 512 // W))
        t = jnp.concatenate(
            [conv_rows(r * RP, RP) for r in range(TH // RP)], axis=1)

    off = jax.lax.dot_general(rw_ref[...], t, (((1,), (0,)), ((), ())),
                              preferred_element_type=jnp.float32) + rb_ref[:, :]
    obj = jax.lax.dot_general(ow_ref[...], t, (((1,), (0,)), ((), ())),
                              preferred_element_type=jnp.float32) + ob_ref[:, :]

    ii = jax.lax.broadcasted_iota(jnp.int32, (1, n), 1)
    cx = (ii % W).astype(jnp.float32) + 0.5
    cy = (ii // W).astype(jnp.float32) + (i * TH).astype(jnp.float32) + 0.5

    ws_all = _BASE_SIZE * np.sqrt(_RATIOS)
    hs_all = _BASE_SIZE / np.sqrt(_RATIOS)

    for a_i in range(_NUM_ANCHORS):
        dx = off[6 * a_i + 0:6 * a_i + 1, :]
        dy = off[6 * a_i + 1:6 * a_i + 2, :]
        dw = off[6 * a_i + 2:6 * a_i + 3, :]
        dh = off[6 * a_i + 3:6 * a_i + 4, :]
        da = off[6 * a_i + 4:6 * a_i + 5, :] * _STD_AB
        db = off[6 * a_i + 5:6 * a_i + 6, :] * _STD_AB
        aw = float(ws_all[a_i])
        ah = float(hs_all[a_i])
        w_ = aw * jnp.exp(jnp.clip(dw, -8.0, 8.0))
        h_ = ah * jnp.exp(jnp.clip(dh, -8.0, 8.0))
        px = cx + dx * aw
        py = cy + dy * ah
        dal = da * w_
        dbe = db * h_
        v1x = px + dal
        v1y = py - h_ / 2.0
        v2x = px + w_ / 2.0
        v2y = py + dbe
        v3x = px - dal
        v3y = py + h_ / 2.0
        v4x = px - w_ / 2.0
        v4y = py - dbe
        x1 = jnp.minimum(jnp.minimum(v1x, v2x), jnp.minimum(v3x, v4x))
        y1 = jnp.minimum(jnp.minimum(v1y, v2y), jnp.minimum(v3y, v4y))
        x2 = jnp.maximum(jnp.maximum(v1x, v2x), jnp.maximum(v3x, v4x))
        y2 = jnp.maximum(jnp.maximum(v1y, v2y), jnp.maximum(v3y, v4y))
        out_ref[a_i, :, :] = jnp.concatenate(
            [v1x, v1y, v2x, v2y, v3x, v3y, v4x, v4y,
             x1, y1, x2, y2, obj[a_i:a_i + 1, :]], axis=0)


def _run_level(x, p, H, W):
    if H * W <= 4096:
        TH = H
    else:
        TH = max(8, 4096 // W)
        while H % TH:
            TH -= 1
    n_tiles = H // TH
    WP = max(128, ((W + 2 + 127) // 128) * 128)

    BH = H + 2 if n_tiles == 1 else ((TH + 2 + 7) // 8) * 8
    HP = H + 2 if n_tiles == 1 else (n_tiles - 1) * TH + BH
    xp = jnp.zeros((256, HP, WP), jnp.float32)
    xp = jax.lax.dynamic_update_slice(xp, x[0], (0, 1, 1))
    ow_pad = jnp.zeros((8, 256), jnp.float32).at[:3].set(p['obj_w'][:, :, 0, 0])
    ob_pad = jnp.zeros((8, 1), jnp.float32).at[:3, 0].set(p['obj_b'])

    if n_tiles == 1:
        x_spec = pl.BlockSpec((256, BH, WP), lambda i: (0, 0, 0))
    else:
        x_spec = pl.BlockSpec((pl.Element(256), pl.Element(BH), pl.Element(WP)),
                              lambda i: (0, i * TH, 0))

    out = pl.pallas_call(
        functools.partial(_conv_decode_kernel, W=W, TH=TH, WP=WP),
        grid=(n_tiles,),
        in_specs=[
            x_spec,
            pl.BlockSpec((256, 2304), lambda i: (0, 0)),
            pl.BlockSpec((256, 1), lambda i: (0, 0)),
            pl.BlockSpec((18, 256), lambda i: (0, 0)),
            pl.BlockSpec((18, 1), lambda i: (0, 0)),
            pl.BlockSpec((8, 256), lambda i: (0, 0)),
            pl.BlockSpec((8, 1), lambda i: (0, 0)),
        ],
        out_specs=pl.BlockSpec((3, 13, TH * W), lambda i: (0, 0, i)),
        out_shape=jax.ShapeDtypeStruct((3, 13, H * W), jnp.float32),
        scratch_shapes=[pltpu.VMEM((256, TH * W), jnp.float32)],
    )(xp, p['conv_w'].transpose(0, 2, 3, 1).reshape(256, 2304),
      p['conv_b'][:, None], p['reg_w'][:, :, 0, 0],
      p['reg_b'][:, None], ow_pad, ob_pad)

    obj = out[:, 12, :].reshape(-1)
    preds = jnp.moveaxis(out[:, :8, :], 1, 2).reshape(-1, 4, 2)
    hbb = jnp.moveaxis(out[:, 8:12, :], 1, 2).reshape(-1, 4)
    return obj, preds, hbb


_NMS_PAD = 2048


_NMS_R = 16
_NMS_C = 128


def _nms_kernel(k_ref, bs_ref, boxes_ref, keep_ref):
    R, C = _NMS_R, _NMS_C
    x1 = boxes_ref[0, :, :]
    y1 = boxes_ref[1, :, :]
    x2 = boxes_ref[2, :, :]
    y2 = boxes_ref[3, :, :]
    areas = jnp.maximum(x2 - x1, 0.0) * jnp.maximum(y2 - y1, 0.0)
    idx = (jax.lax.broadcasted_iota(jnp.int32, (R, C), 0) * C
           + jax.lax.broadcasted_iota(jnp.int32, (R, C), 1)).astype(jnp.float32)

    U = 8

    def body(blk, keep):
        i0 = blk * U
        sups = []
        for u in range(U):
            i = i0 + u
            fi = i.astype(jnp.float32)
            bx1 = bs_ref[0, i]
            by1 = bs_ref[1, i]
            bx2 = bs_ref[2, i]
            by2 = bs_ref[3, i]
            ar_i = jnp.maximum(bx2 - bx1, 0.0) * jnp.maximum(by2 - by1, 0.0)
            xx1 = jnp.maximum(x1, bx1)
            yy1 = jnp.maximum(y1, by1)
            xx2 = jnp.minimum(x2, bx2)
            yy2 = jnp.minimum(y2, by2)
            inter = jnp.maximum(xx2 - xx1, 0.0) * jnp.maximum(yy2 - yy1, 0.0)
            iou = inter / (areas + ar_i - inter + 1e-9)
            sups.append(((iou > 0.5) & (idx > fi)).astype(jnp.float32))
        for u in range(U):
            fi = (i0 + u).astype(jnp.float32)
            ki = jnp.sum(keep * (idx == fi).astype(jnp.float32))
            keep = keep * (1.0 - ki * sups[u])
        return keep

    nblk = (k_ref[0] + U - 1) // U
    keep_ref[...] = jax.lax.fori_loop(0, nblk, body,
                                      jnp.ones((R, C), jnp.float32))


def _nms(boxes, k):
    NPAD = _NMS_R * _NMS_C
    bpad = jnp.zeros((NPAD, 4), jnp.float32).at[:k].set(boxes).T
    keep = pl.pallas_call(
        _nms_kernel,
        in_specs=[
            pl.BlockSpec(memory_space=pltpu.SMEM),
            pl.BlockSpec(memory_space=pltpu.SMEM),
            pl.BlockSpec((4, _NMS_R, _NMS_C), lambda: (0, 0, 0)),
        ],
        out_shape=jax.ShapeDtypeStruct((_NMS_R, _NMS_C), jnp.float32),
    )(jnp.array([k], jnp.int32), bpad, bpad.reshape(4, _NMS_R, _NMS_C))
    return keep.reshape(-1)[:k] > 0.5


def kernel(x0, x1, x2, x3, x4, params):
    feats = [x0, x1, x2, x3, x4]
    merged_props = []
    merged_scores = []
    for lvl in range(5):
        x = feats[lvl]
        p = params[str(lvl)]
        _, _, H, W = x.shape
        obj, preds, hbb = _run_level(x, p, H, W)
        k = min(2000, obj.shape[0])
        scores, ti = jax.lax.top_k(obj, k)
        keep = _nms(hbb[ti], k)
        s = jnp.where(keep, scores, -1e9)
        merged_props.append(preds[ti][None])
        merged_scores.append(s[None])
    props = jnp.concatenate(merged_props, axis=1)
    scores = jnp.concatenate(merged_scores, axis=1)
    kk = min(1000, scores.shape[1])
    _, ti = jax.lax.top_k(scores[0], kk)
    return props[:, ti]


# speedup vs baseline: 68.1105x; 2.2338x over previous
import functools

import jax
import jax.numpy as jnp
import numpy as np
from jax.experimental import pallas as pl
from jax.experimental.pallas import tpu as pltpu

_NUM_ANCHORS = 3
_RATIOS = np.array([0.5, 1.0, 2.0])
_BASE_SIZE = 8.0
_STD_AB = 0.5


def _conv_decode_kernel(xp_ref, w_ref, b_ref, rw_ref, rb_ref, ow_ref, ob_ref,
                        out_ref, t_scr, *, W, TH, WP):
    i = pl.program_id(0)
    n = TH * W

    def conv_rows(r, RP):
        xr = [xp_ref[:, pl.ds(r + j, 1), :].reshape(256, WP)
              for j in range(RP + 2)]
        ks = []
        for kh in range(3):
            for kw in range(3):
                if RP == 1:
                    ks.append(xr[kh][:, kw:kw + W])
                else:
                    ks.append(jnp.concatenate(
                        [xr[j + kh][:, kw:kw + W] for j in range(RP)], axis=1))
        a = jnp.concatenate(ks, axis=0)
        acc = jax.lax.dot_general(
            w_ref[...], a, (((1,), (0,)), ((), ())),
            preferred_element_type=jnp.float32)
        return acc + b_ref[:, :]

    if W >= 128:
        RP = 4
        def row_body(it, carry):
            r = it * RP
            t_scr[:, pl.ds(r * W, RP * W)] = conv_rows(r, RP)
            return carry

        jax.lax.fori_loop(0, TH // RP, row_body, 0, unroll=False)
        t = t_scr[...]
    else:
        RP = min(TH, max(1, 512 // W))
        t = jnp.concatenate(
            [conv_rows(r * RP, RP) for r in range(TH // RP)], axis=1)

    off = jax.lax.dot_general(rw_ref[...], t, (((1,), (0,)), ((), ())),
                              preferred_element_type=jnp.float32) + rb_ref[:, :]
    obj = jax.lax.dot_general(ow_ref[...], t, (((1,), (0,)), ((), ())),
                              preferred_element_type=jnp.float32) + ob_ref[:, :]

    ii = jax.lax.broadcasted_iota(jnp.int32, (1, n), 1)
    cx = (ii % W).astype(jnp.float32) + 0.5
    cy = (ii // W).astype(jnp.float32) + (i * TH).astype(jnp.float32) + 0.5

    ws_all = _BASE_SIZE * np.sqrt(_RATIOS)
    hs_all = _BASE_SIZE / np.sqrt(_RATIOS)

    for a_i in range(_NUM_ANCHORS):
        dx = off[6 * a_i + 0:6 * a_i + 1, :]
        dy = off[6 * a_i + 1:6 * a_i + 2, :]
        dw = off[6 * a_i + 2:6 * a_i + 3, :]
        dh = off[6 * a_i + 3:6 * a_i + 4, :]
        da = off[6 * a_i + 4:6 * a_i + 5, :] * _STD_AB
        db = off[6 * a_i + 5:6 * a_i + 6, :] * _STD_AB
        aw = float(ws_all[a_i])
        ah = float(hs_all[a_i])
        w_ = aw * jnp.exp(jnp.clip(dw, -8.0, 8.0))
        h_ = ah * jnp.exp(jnp.clip(dh, -8.0, 8.0))
        px = cx + dx * aw
        py = cy + dy * ah
        dal = da * w_
        dbe = db * h_
        v1x = px + dal
        v1y = py - h_ / 2.0
        v2x = px + w_ / 2.0
        v2y = py + dbe
        v3x = px - dal
        v3y = py + h_ / 2.0
        v4x = px - w_ / 2.0
        v4y = py - dbe
        x1 = jnp.minimum(jnp.minimum(v1x, v2x), jnp.minimum(v3x, v4x))
        y1 = jnp.minimum(jnp.minimum(v1y, v2y), jnp.minimum(v3y, v4y))
        x2 = jnp.maximum(jnp.maximum(v1x, v2x), jnp.maximum(v3x, v4x))
        y2 = jnp.maximum(jnp.maximum(v1y, v2y), jnp.maximum(v3y, v4y))
        out_ref[a_i, :, :] = jnp.concatenate(
            [v1x, v1y, v2x, v2y, v3x, v3y, v4x, v4y,
             x1, y1, x2, y2, obj[a_i:a_i + 1, :]], axis=0)


def _run_level(x, p, H, W):
    if H * W <= 4096:
        TH = H
    else:
        TH = max(8, 4096 // W)
        while H % TH:
            TH -= 1
    n_tiles = H // TH
    WP = max(128, ((W + 2 + 127) // 128) * 128)

    BH = H + 2 if n_tiles == 1 else ((TH + 2 + 7) // 8) * 8
    HP = H + 2 if n_tiles == 1 else (n_tiles - 1) * TH + BH
    xp = jnp.zeros((256, HP, WP), jnp.float32)
    xp = jax.lax.dynamic_update_slice(xp, x[0], (0, 1, 1))
    ow_pad = jnp.zeros((8, 256), jnp.float32).at[:3].set(p['obj_w'][:, :, 0, 0])
    ob_pad = jnp.zeros((8, 1), jnp.float32).at[:3, 0].set(p['obj_b'])

    if n_tiles == 1:
        x_spec = pl.BlockSpec((256, BH, WP), lambda i: (0, 0, 0))
    else:
        x_spec = pl.BlockSpec((pl.Element(256), pl.Element(BH), pl.Element(WP)),
                              lambda i: (0, i * TH, 0))

    out = pl.pallas_call(
        functools.partial(_conv_decode_kernel, W=W, TH=TH, WP=WP),
        grid=(n_tiles,),
        in_specs=[
            x_spec,
            pl.BlockSpec((256, 2304), lambda i: (0, 0)),
            pl.BlockSpec((256, 1), lambda i: (0, 0)),
            pl.BlockSpec((18, 256), lambda i: (0, 0)),
            pl.BlockSpec((18, 1), lambda i: (0, 0)),
            pl.BlockSpec((8, 256), lambda i: (0, 0)),
            pl.BlockSpec((8, 1), lambda i: (0, 0)),
        ],
        out_specs=pl.BlockSpec((3, 13, TH * W), lambda i: (0, 0, i)),
        out_shape=jax.ShapeDtypeStruct((3, 13, H * W), jnp.float32),
        scratch_shapes=[pltpu.VMEM((256, TH * W), jnp.float32)],
    )(xp, p['conv_w'].transpose(0, 2, 3, 1).reshape(256, 2304),
      p['conv_b'][:, None], p['reg_w'][:, :, 0, 0],
      p['reg_b'][:, None], ow_pad, ob_pad)

    obj = out[:, 12, :].reshape(-1)
    preds = jnp.moveaxis(out[:, :8, :], 1, 2).reshape(-1, 4, 2)
    hbb = jnp.moveaxis(out[:, 8:12, :], 1, 2).reshape(-1, 4)
    return obj, preds, hbb


_NMS_PAD = 2048


_NMS_PAD = 2048


def _nms_kernel(bj_ref, bi_ref, keep_ref, m_scr):
    N = _NMS_PAD
    CH = 128
    x1j = bj_ref[0:1, :]
    y1j = bj_ref[1:2, :]
    x2j = bj_ref[2:3, :]
    y2j = bj_ref[3:4, :]
    areas_j = jnp.maximum(x2j - x1j, 0.0) * jnp.maximum(y2j - y1j, 0.0)
    jj = jax.lax.broadcasted_iota(jnp.int32, (1, N), 1)

    def mchunk(ri, carry):
        r0 = ri * CH
        x1i = bi_ref[pl.ds(r0, CH), 0:1]
        y1i = bi_ref[pl.ds(r0, CH), 1:2]
        x2i = bi_ref[pl.ds(r0, CH), 2:3]
        y2i = bi_ref[pl.ds(r0, CH), 3:4]
        areas_i = jnp.maximum(x2i - x1i, 0.0) * jnp.maximum(y2i - y1i, 0.0)
        ii = jax.lax.broadcasted_iota(jnp.int32, (CH, 1), 0) + r0
        xx1 = jnp.maximum(x1i, x1j)
        yy1 = jnp.maximum(y1i, y1j)
        xx2 = jnp.minimum(x2i, x2j)
        yy2 = jnp.minimum(y2i, y2j)
        inter = jnp.maximum(xx2 - xx1, 0.0) * jnp.maximum(yy2 - yy1, 0.0)
        iou = inter / (areas_i + areas_j - inter + 1e-9)
        m = ((iou > 0.5) & (ii < jj)).astype(jnp.bfloat16)
        m_scr[pl.ds(r0, CH), :] = m
        return carry

    jax.lax.fori_loop(0, N // CH, mchunk, 0, unroll=False)

    def cond(c):
        return c[1]

    def body(c):
        k, _ = c
        s = jax.lax.dot_general(
            k.astype(jnp.bfloat16), m_scr[...], (((1,), (0,)), ((), ())),
            preferred_element_type=jnp.float32)
        knew = (s < 0.5).astype(jnp.float32)
        return (knew, jnp.max(jnp.abs(knew - k)) > 0.0)

    k, _ = jax.lax.while_loop(cond, body,
                              (jnp.ones((1, N), jnp.float32), True))
    keep_ref[...] = k


def _nms(boxes, k):
    bpad = jnp.zeros((_NMS_PAD, 4), jnp.float32).at[:k].set(boxes)
    keep = pl.pallas_call(
        _nms_kernel,
        in_specs=[
            pl.BlockSpec((4, _NMS_PAD), lambda: (0, 0)),
            pl.BlockSpec((_NMS_PAD, 4), lambda: (0, 0)),
        ],
        out_shape=jax.ShapeDtypeStruct((1, _NMS_PAD), jnp.float32),
        scratch_shapes=[pltpu.VMEM((_NMS_PAD, _NMS_PAD), jnp.bfloat16)],
    )(bpad.T, bpad)
    return keep[0, :k] > 0.5


def kernel(x0, x1, x2, x3, x4, params):
    feats = [x0, x1, x2, x3, x4]
    merged_props = []
    merged_scores = []
    for lvl in range(5):
        x = feats[lvl]
        p = params[str(lvl)]
        _, _, H, W = x.shape
        obj, preds, hbb = _run_level(x, p, H, W)
        k = min(2000, obj.shape[0])
        scores, ti = jax.lax.top_k(obj, k)
        keep = _nms(hbb[ti], k)
        s = jnp.where(keep, scores, -1e9)
        merged_props.append(preds[ti][None])
        merged_scores.append(s[None])
    props = jnp.concatenate(merged_props, axis=1)
    scores = jnp.concatenate(merged_scores, axis=1)
    kk = min(1000, scores.shape[1])
    _, ti = jax.lax.top_k(scores[0], kk)
    return props[:, ti]


# speedup vs baseline: 70.9596x; 1.0418x over previous
import functools

import jax
import jax.numpy as jnp
import numpy as np
from jax.experimental import pallas as pl
from jax.experimental.pallas import tpu as pltpu

_NUM_ANCHORS = 3
_RATIOS = np.array([0.5, 1.0, 2.0])
_BASE_SIZE = 8.0
_STD_AB = 0.5


def _conv_decode_kernel(xp_ref, w_ref, b_ref, rw_ref, rb_ref, ow_ref, ob_ref,
                        out_ref, t_scr, *, W, TH, WP):
    i = pl.program_id(0)
    n = TH * W

    def tap(xrow, kw):
        if WP > W:
            return xrow[:, kw:kw + W]
        if kw == 0:
            return jnp.concatenate(
                [jnp.zeros((256, 1), jnp.float32), xrow[:, :W - 1]], axis=1)
        if kw == 1:
            return xrow
        return jnp.concatenate(
            [xrow[:, 1:], jnp.zeros((256, 1), jnp.float32)], axis=1)

    def conv_rows(r, RP):
        xr = [xp_ref[:, pl.ds(r + j, 1), :].reshape(256, WP)
              for j in range(RP + 2)]
        ks = []
        for kh in range(3):
            for kw in range(3):
                if RP == 1:
                    ks.append(tap(xr[kh], kw))
                else:
                    ks.append(jnp.concatenate(
                        [tap(xr[j + kh], kw) for j in range(RP)], axis=1))
        a = jnp.concatenate(ks, axis=0)
        acc = jax.lax.dot_general(
            w_ref[...], a, (((1,), (0,)), ((), ())),
            preferred_element_type=jnp.float32)
        return acc + b_ref[:, :]

    if W >= 128:
        RP = 4
        def row_body(it, carry):
            r = it * RP
            t_scr[:, pl.ds(r * W, RP * W)] = conv_rows(r, RP)
            return carry

        jax.lax.fori_loop(0, TH // RP, row_body, 0, unroll=False)
        t = t_scr[...]
    else:
        RP = min(TH, max(1, 512 // W))
        t = jnp.concatenate(
            [conv_rows(r * RP, RP) for r in range(TH // RP)], axis=1)

    off = jax.lax.dot_general(rw_ref[...], t, (((1,), (0,)), ((), ())),
                              preferred_element_type=jnp.float32) + rb_ref[:, :]
    obj = jax.lax.dot_general(ow_ref[...], t, (((1,), (0,)), ((), ())),
                              preferred_element_type=jnp.float32) + ob_ref[:, :]

    ii = jax.lax.broadcasted_iota(jnp.int32, (1, n), 1)
    cx = (ii % W).astype(jnp.float32) + 0.5
    cy = (ii // W).astype(jnp.float32) + (i * TH).astype(jnp.float32) + 0.5

    ws_all = _BASE_SIZE * np.sqrt(_RATIOS)
    hs_all = _BASE_SIZE / np.sqrt(_RATIOS)

    for a_i in range(_NUM_ANCHORS):
        dx = off[6 * a_i + 0:6 * a_i + 1, :]
        dy = off[6 * a_i + 1:6 * a_i + 2, :]
        dw = off[6 * a_i + 2:6 * a_i + 3, :]
        dh = off[6 * a_i + 3:6 * a_i + 4, :]
        da = off[6 * a_i + 4:6 * a_i + 5, :] * _STD_AB
        db = off[6 * a_i + 5:6 * a_i + 6, :] * _STD_AB
        aw = float(ws_all[a_i])
        ah = float(hs_all[a_i])
        w_ = aw * jnp.exp(jnp.clip(dw, -8.0, 8.0))
        h_ = ah * jnp.exp(jnp.clip(dh, -8.0, 8.0))
        px = cx + dx * aw
        py = cy + dy * ah
        dal = da * w_
        dbe = db * h_
        v1x = px + dal
        v1y = py - h_ / 2.0
        v2x = px + w_ / 2.0
        v2y = py + dbe
        v3x = px - dal
        v3y = py + h_ / 2.0
        v4x = px - w_ / 2.0
        v4y = py - dbe
        x1 = jnp.minimum(jnp.minimum(v1x, v2x), jnp.minimum(v3x, v4x))
        y1 = jnp.minimum(jnp.minimum(v1y, v2y), jnp.minimum(v3y, v4y))
        x2 = jnp.maximum(jnp.maximum(v1x, v2x), jnp.maximum(v3x, v4x))
        y2 = jnp.maximum(jnp.maximum(v1y, v2y), jnp.maximum(v3y, v4y))
        out_ref[a_i, :, :] = jnp.concatenate(
            [v1x, v1y, v2x, v2y, v3x, v3y, v4x, v4y,
             x1, y1, x2, y2, obj[a_i:a_i + 1, :]], axis=0)


def _run_level(x, p, H, W):
    if H * W <= 4096:
        TH = H
    else:
        TH = max(8, 4096 // W)
        while H % TH:
            TH -= 1
    n_tiles = H // TH
    WP = W if W >= 128 else 128

    BH = H + 2 if n_tiles == 1 else ((TH + 2 + 7) // 8) * 8
    HP = H + 2 if n_tiles == 1 else (n_tiles - 1) * TH + BH
    xp = jnp.zeros((256, HP, WP), jnp.float32)
    xp = jax.lax.dynamic_update_slice(xp, x[0], (0, 1, 1 if WP > W else 0))
    ow_pad = jnp.zeros((8, 256), jnp.float32).at[:3].set(p['obj_w'][:, :, 0, 0])
    ob_pad = jnp.zeros((8, 1), jnp.float32).at[:3, 0].set(p['obj_b'])

    if n_tiles == 1:
        x_spec = pl.BlockSpec((256, BH, WP), lambda i: (0, 0, 0))
    else:
        x_spec = pl.BlockSpec((pl.Element(256), pl.Element(BH), pl.Element(WP)),
                              lambda i: (0, i * TH, 0))

    out = pl.pallas_call(
        functools.partial(_conv_decode_kernel, W=W, TH=TH, WP=WP),
        grid=(n_tiles,),
        in_specs=[
            x_spec,
            pl.BlockSpec((256, 2304), lambda i: (0, 0)),
            pl.BlockSpec((256, 1), lambda i: (0, 0)),
            pl.BlockSpec((18, 256), lambda i: (0, 0)),
            pl.BlockSpec((18, 1), lambda i: (0, 0)),
            pl.BlockSpec((8, 256), lambda i: (0, 0)),
            pl.BlockSpec((8, 1), lambda i: (0, 0)),
        ],
        out_specs=pl.BlockSpec((3, 13, TH * W), lambda i: (0, 0, i)),
        out_shape=jax.ShapeDtypeStruct((3, 13, H * W), jnp.float32),
        scratch_shapes=[pltpu.VMEM((256, TH * W), jnp.float32)],
    )(xp, p['conv_w'].transpose(0, 2, 3, 1).reshape(256, 2304),
      p['conv_b'][:, None], p['reg_w'][:, :, 0, 0],
      p['reg_b'][:, None], ow_pad, ob_pad)

    obj = out[:, 12, :].reshape(-1)
    preds = jnp.moveaxis(out[:, :8, :], 1, 2).reshape(-1, 4, 2)
    hbb = jnp.moveaxis(out[:, 8:12, :], 1, 2).reshape(-1, 4)
    return obj, preds, hbb


_NMS_PAD = 2048


_NMS_PAD = 2048


def _nms_kernel(bj_ref, bi_ref, keep_ref, m_scr):
    N = _NMS_PAD
    CH = 128
    x1j = bj_ref[0:1, :]
    y1j = bj_ref[1:2, :]
    x2j = bj_ref[2:3, :]
    y2j = bj_ref[3:4, :]
    areas_j = jnp.maximum(x2j - x1j, 0.0) * jnp.maximum(y2j - y1j, 0.0)
    jj = jax.lax.broadcasted_iota(jnp.int32, (1, N), 1)

    def mchunk(ri, carry):
        r0 = ri * CH
        x1i = bi_ref[pl.ds(r0, CH), 0:1]
        y1i = bi_ref[pl.ds(r0, CH), 1:2]
        x2i = bi_ref[pl.ds(r0, CH), 2:3]
        y2i = bi_ref[pl.ds(r0, CH), 3:4]
        areas_i = jnp.maximum(x2i - x1i, 0.0) * jnp.maximum(y2i - y1i, 0.0)
        ii = jax.lax.broadcasted_iota(jnp.int32, (CH, 1), 0) + r0
        xx1 = jnp.maximum(x1i, x1j)
        yy1 = jnp.maximum(y1i, y1j)
        xx2 = jnp.minimum(x2i, x2j)
        yy2 = jnp.minimum(y2i, y2j)
        inter = jnp.maximum(xx2 - xx1, 0.0) * jnp.maximum(yy2 - yy1, 0.0)
        iou = inter / (areas_i + areas_j - inter + 1e-9)
        m = ((iou > 0.5) & (ii < jj)).astype(jnp.bfloat16)
        m_scr[pl.ds(r0, CH), :] = m
        return carry

    jax.lax.fori_loop(0, N // CH, mchunk, 0, unroll=False)

    def cond(c):
        return c[1]

    def body(c):
        k, _ = c
        s = jax.lax.dot_general(
            k.astype(jnp.bfloat16), m_scr[...], (((1,), (0,)), ((), ())),
            preferred_element_type=jnp.float32)
        knew = (s < 0.5).astype(jnp.float32)
        return (knew, jnp.max(jnp.abs(knew - k)) > 0.0)

    k, _ = jax.lax.while_loop(cond, body,
                              (jnp.ones((1, N), jnp.float32), True))
    keep_ref[...] = k


def _nms(boxes, k):
    bpad = jnp.zeros((_NMS_PAD, 4), jnp.float32).at[:k].set(boxes)
    keep = pl.pallas_call(
        _nms_kernel,
        in_specs=[
            pl.BlockSpec((4, _NMS_PAD), lambda: (0, 0)),
            pl.BlockSpec((_NMS_PAD, 4), lambda: (0, 0)),
        ],
        out_shape=jax.ShapeDtypeStruct((1, _NMS_PAD), jnp.float32),
        scratch_shapes=[pltpu.VMEM((_NMS_PAD, _NMS_PAD), jnp.bfloat16)],
    )(bpad.T, bpad)
    return keep[0, :k] > 0.5


def kernel(x0, x1, x2, x3, x4, params):
    feats = [x0, x1, x2, x3, x4]
    merged_props = []
    merged_scores = []
    for lvl in range(5):
        x = feats[lvl]
        p = params[str(lvl)]
        _, _, H, W = x.shape
        obj, preds, hbb = _run_level(x, p, H, W)
        k = min(2000, obj.shape[0])
        scores, ti = jax.lax.top_k(obj, k)
        keep = _nms(hbb[ti], k)
        s = jnp.where(keep, scores, -1e9)
        merged_props.append(preds[ti][None])
        merged_scores.append(s[None])
    props = jnp.concatenate(merged_props, axis=1)
    scores = jnp.concatenate(merged_scores, axis=1)
    kk = min(1000, scores.shape[1])
    _, ti = jax.lax.top_k(scores[0], kk)
    return props[:, ti]
